# baseline (speedup 1.0000x reference)
import functools

import jax
import jax.numpy as jnp
from jax import lax
from jax.experimental import pallas as pl
from jax.experimental.pallas import tpu as pltpu
from jax.experimental.pallas import tpu_sc as plsc

VOCAB = 1000000
EMBED_DIM = 32
BATCH = 4096
HIST = 200
TOTAL = BATCH * HIST

NUM_CORES = 2
NUM_SUBCORES = 16
NW = NUM_CORES * NUM_SUBCORES

ABLK = 512
N_ABLK_FULL = VOCAB // ABLK
A_LOOP = 61
A_TAIL = VOCAB - N_ABLK_FULL * ABLK

CHUNK = 1024
CPW = TOTAL // (CHUNK * NW)
CGRP = BATCH // CHUNK
G = EMBED_DIM // 8
OUT_ROWS = TOTAL * EMBED_DIM // 128


def _make_relayout_kernel():
    mesh = plsc.VectorSubcoreMesh(core_axis_name="c", subcore_axis_name="s")

    @functools.partial(
        pl.kernel,
        mesh=mesh,
        out_type=jax.ShapeDtypeStruct((VOCAB // 4, 128), jnp.float32),
        scratch_types=[
            pltpu.VMEM((EMBED_DIM, ABLK + 8), jnp.float32),
            pltpu.VMEM((EMBED_DIM, ABLK + 8), jnp.float32),
            pltpu.VMEM((128, 128), jnp.float32),
            pltpu.VMEM((128, 128), jnp.float32),
            pltpu.SemaphoreType.DMA,
            pltpu.SemaphoreType.DMA,
            pltpu.SemaphoreType.DMA,
            pltpu.SemaphoreType.DMA,
        ],
        compiler_params=pltpu.CompilerParams(
            use_tc_tiling_on_sc=True, needs_layout_passes=False),
    )
    def relayout_kernel(tt_hbm, tail_hbm, out_hbm, in0, in1, tr0, tr1,
                        si0, si1, sw0, sw1):
        wid = lax.axis_index("s") * NUM_CORES + lax.axis_index("c")
        ins = (in0, in1)
        trs = (tr0, tr1)
        sis = (si0, si1)
        sws = (sw0, sw1)
        lanes = lax.iota(jnp.int32, 16)
        evec = [lanes + 16 * half for half in range(2)]

        def blk(j):
            return wid + NW * j

        def in_start(j, p):
            pltpu.async_copy(
                tt_hbm.at[:, pl.ds(blk(j) * ABLK, ABLK)],
                ins[p].at[:, pl.ds(0, ABLK)], sis[p])

        def in_wait(p):
            pltpu.make_async_copy(
                tt_hbm.at[:, pl.ds(0, ABLK)],
                ins[p].at[:, pl.ds(0, ABLK)], sis[p]).wait()

        def out_start(j, p):
            pltpu.async_copy(
                trs[p], out_hbm.at[pl.ds(blk(j) * 128, 128), :], sws[p])

        def out_wait(p):
            pltpu.make_async_copy(
                trs[p], out_hbm.at[pl.ds(0, 128), :], sws[p]).wait()

        rvec_pat = lanes // 4
        cvec_pat = (lanes % 4) * 32
        def transpose_block(src, dst, nrows):
            @plsc.parallel_loop(0, nrows // 4, unroll=4)
            def q_body(q):
                j0 = q * 16
                rvec = rvec_pat + (4 * q)
                for e in range(EMBED_DIM):
                    v = src[e, pl.ds(j0, 16)]
                    plsc.store_scatter(dst, [rvec, cvec_pat + e], v)

        def step(j, p, first):
            with jax.named_scope("a_inwait"):
                in_wait(p)
            in_start(j + 1, 1 - p)
            if not first:
                with jax.named_scope("a_outwait"):
                    out_wait(p)
            with jax.named_scope("a_tr"):
                transpose_block(ins[p], trs[p], 128)
            out_start(j, p)

        in_start(0, 0)
        step(0, 0, True)
        step(1, 1, True)

        def pair_body(j2, carry):
            j = 2 + 2 * j2
            step(j, 0, False)
            step(j + 1, 1, False)
            return carry
        lax.fori_loop(0, (A_LOOP - 3) // 2, pair_body, 0)

        in_wait(0)
        out_wait(0)
        transpose_block(in0, tr0, 128)
        out_start(A_LOOP - 1, 0)
        out_wait(1)
        out_wait(0)

        @pl.when(wid == 0)
        def _():
            k = N_ABLK_FULL - 1
            pltpu.sync_copy(tt_hbm.at[:, pl.ds(k * ABLK, ABLK)],
                            in0.at[:, pl.ds(0, ABLK)])
            transpose_block(in0, tr0, 128)
            pltpu.sync_copy(tr0, out_hbm.at[pl.ds(k * 128, 128), :])

        @pl.when(wid == 1)
        def _():
            nrows = A_TAIL // 4
            pltpu.sync_copy(tail_hbm, tr1.at[pl.ds(0, nrows), :])
            pltpu.sync_copy(
                tr1.at[pl.ds(0, nrows), :],
                out_hbm.at[pl.ds((N_ABLK_FULL * ABLK) // 4, nrows), :])

    return relayout_kernel


def _make_gather_kernel():
    mesh = plsc.VectorSubcoreMesh(core_axis_name="c", subcore_axis_name="s")

    @functools.partial(
        pl.kernel,
        mesh=mesh,
        out_type=jax.ShapeDtypeStruct((OUT_ROWS, 128), jnp.float32),
        scratch_types=[
            pltpu.VMEM((CHUNK,), jnp.int32),
            pltpu.VMEM((CHUNK,), jnp.int32),
            pltpu.VMEM((CHUNK, EMBED_DIM), jnp.float32),
            pltpu.VMEM((CHUNK, EMBED_DIM), jnp.float32),
            pltpu.VMEM((G * 64, 128), jnp.float32),
            pltpu.SemaphoreType.DMA,
            pltpu.SemaphoreType.DMA,
            pltpu.SemaphoreType.DMA,
        ],
        compiler_params=pltpu.CompilerParams(
            use_tc_tiling_on_sc=False, needs_layout_passes=False),
    )
    def gather_kernel(table_hbm, idxt_hbm, out_hbm, idx0, idx1,
                      rows0, rows1, slab, sg0, sg1, sw):
        wid = lax.axis_index("s") * NUM_CORES + lax.axis_index("c")
        idxs = (idx0, idx1)
        rows = (rows0, rows1)
        sgs = (sg0, sg1)
        lanes = lax.iota(jnp.int32, 16)

        def chunk_hcg(k):
            kk = wid * CPW + k
            return kk // CGRP, kk % CGRP

        def idx_copy(k, p):
            h, cg = chunk_hcg(k)
            pltpu.sync_copy(
                idxt_hbm.at[pl.ds(h * BATCH + cg * CHUNK, CHUNK)], idxs[p])

        def gather_start(p):
            pltpu.async_copy(table_hbm.at[idxs[p]], rows[p], sgs[p])

        def gather_wait(p):
            pltpu.make_async_copy(
                table_hbm.at[pl.ds(0, CHUNK)], rows[p], sgs[p]).wait()

        def write_start(k):
            h, cg = chunk_hcg(k)
            for g in range(G):
                r0 = ((h * G + g) * (BATCH // 128) + cg * 8) * 8
                pltpu.async_copy(slab.at[pl.ds(g * 64, 64), :],
                                 out_hbm.at[pl.ds(r0, 64), :], sw)

        def write_wait():
            for g in range(G):
                pltpu.make_async_copy(slab.at[pl.ds(g * 64, 64), :],
                                      out_hbm.at[pl.ds(0, 64), :], sw).wait()

        def transpose_chunk(p):
            src = rows[p]

            @plsc.parallel_loop(0, 256, unroll=8)
            def q_body(q):
                g = q // 64
                rm = q % 64
                cc = rm // 8
                e = rm % 8
                col = jnp.full((16,), g * 8 + e, jnp.int32)
                base = cc * 128
                for t in range(8):
                    v = plsc.load_gather(src, [(base + 16 * t) + lanes, col])
                    slab[q, pl.ds(16 * t, 16)] = v

        def step(k, p, first):
            with jax.named_scope("b_gwait"):
                gather_wait(p)
            gather_start(1 - p)
            with jax.named_scope("b_idxcopy"):
                idx_copy(jnp.minimum(k + 2, CPW - 1), p)
            if not first:
                with jax.named_scope("b_wwait"):
                    write_wait()
            with jax.named_scope("b_tr"):
                transpose_chunk(p)
            write_start(k)

        idx_copy(0, 0)
        gather_start(0)
        idx_copy(1, 1)
        step(0, 0, True)

        def pair_body(k2, carry):
            k = 1 + 2 * k2
            step(k, 1, False)
            step(k + 1, 0, False)
            return carry
        lax.fori_loop(0, (CPW - 1) // 2, pair_body, 0)

        write_wait()
        gather_wait(1)

    return gather_kernel


_relayout = _make_relayout_kernel()
_gather = _make_gather_kernel()


@jax.jit
def kernel(word_indices, table):
    tt = table.T
    tail16 = lax.slice(
        table, (N_ABLK_FULL * ABLK, 0), (VOCAB, EMBED_DIM)
    ).reshape(A_TAIL // 4, 128)
    tabr = _relayout(tt, tail16)
    tab_lin = tabr.reshape(VOCAB, EMBED_DIM)
    idxt_flat = word_indices.T.reshape(TOTAL).astype(jnp.int32)
    out = _gather(tab_lin, idxt_flat)
    out = out.reshape(HIST, G, BATCH // 128, 8, 128)
    out = out.transpose(2, 4, 0, 1, 3)
    return out.reshape(BATCH, HIST, EMBED_DIM)


# speedup vs baseline: 1.0986x; 1.0986x over previous
import functools

import jax
import jax.numpy as jnp
from jax import lax
from jax.experimental import pallas as pl
from jax.experimental.pallas import tpu as pltpu
from jax.experimental.pallas import tpu_sc as plsc

VOCAB = 1000000
EMBED_DIM = 32
BATCH = 4096
HIST = 200
TOTAL = BATCH * HIST

NUM_CORES = 2
NUM_SUBCORES = 16
NW = NUM_CORES * NUM_SUBCORES

ABLK = 512
N_ABLK_FULL = VOCAB // ABLK
A_LOOP = 61
A_TAIL = VOCAB - N_ABLK_FULL * ABLK

CHUNK = 1024
CPW = TOTAL // (CHUNK * NW)
CGRP = BATCH // CHUNK
G = EMBED_DIM // 8
OUT_ROWS = TOTAL * EMBED_DIM // 128


def _make_relayout_kernel():
    mesh = plsc.VectorSubcoreMesh(core_axis_name="c", subcore_axis_name="s")

    @functools.partial(
        pl.kernel,
        mesh=mesh,
        out_type=jax.ShapeDtypeStruct((VOCAB // 4, 128), jnp.float32),
        scratch_types=[
            pltpu.VMEM((EMBED_DIM, ABLK + 8), jnp.float32),
            pltpu.VMEM((EMBED_DIM, ABLK + 8), jnp.float32),
            pltpu.VMEM((128, 128), jnp.float32),
            pltpu.VMEM((128, 128), jnp.float32),
            pltpu.SemaphoreType.DMA,
            pltpu.SemaphoreType.DMA,
            pltpu.SemaphoreType.DMA,
            pltpu.SemaphoreType.DMA,
        ],
        compiler_params=pltpu.CompilerParams(
            use_tc_tiling_on_sc=True, needs_layout_passes=False),
    )
    def relayout_kernel(tt_hbm, tail_hbm, out_hbm, in0, in1, tr0, tr1,
                        si0, si1, sw0, sw1):
        wid = lax.axis_index("s") * NUM_CORES + lax.axis_index("c")
        ins = (in0, in1)
        trs = (tr0, tr1)
        sis = (si0, si1)
        sws = (sw0, sw1)
        lanes = lax.iota(jnp.int32, 16)
        evec = [lanes + 16 * half for half in range(2)]

        def blk(j):
            return wid + NW * j

        def in_start(j, p):
            pltpu.async_copy(
                tt_hbm.at[:, pl.ds(blk(j) * ABLK, ABLK)],
                ins[p].at[:, pl.ds(0, ABLK)], sis[p])

        def in_wait(p):
            pltpu.make_async_copy(
                tt_hbm.at[:, pl.ds(0, ABLK)],
                ins[p].at[:, pl.ds(0, ABLK)], sis[p]).wait()

        def out_start(j, p):
            pltpu.async_copy(
                trs[p], out_hbm.at[pl.ds(blk(j) * 128, 128), :], sws[p])

        def out_wait(p):
            pltpu.make_async_copy(
                trs[p], out_hbm.at[pl.ds(0, 128), :], sws[p]).wait()

        def transpose_block(src, dst, nrows):
            @plsc.parallel_loop(0, nrows, unroll=8)
            def row_body(r):
                for t in range(8):
                    col = jnp.full((16,), r * 4 + t // 2, jnp.int32)
                    v = plsc.load_gather(src, [evec[t % 2], col])
                    dst[r, pl.ds(16 * t, 16)] = v

        def step(j, p, first):
            with jax.named_scope("a_inwait"):
                in_wait(p)
            in_start(j + 1, 1 - p)
            if not first:
                with jax.named_scope("a_outwait"):
                    out_wait(p)
            with jax.named_scope("a_tr"):
                transpose_block(ins[p], trs[p], 128)
            out_start(j, p)

        in_start(0, 0)
        step(0, 0, True)
        step(1, 1, True)

        def pair_body(j2, carry):
            j = 2 + 2 * j2
            step(j, 0, False)
            step(j + 1, 1, False)
            return carry
        lax.fori_loop(0, (A_LOOP - 3) // 2, pair_body, 0)

        in_wait(0)
        out_wait(0)
        transpose_block(in0, tr0, 128)
        out_start(A_LOOP - 1, 0)
        out_wait(1)
        out_wait(0)

        @pl.when(wid == 0)
        def _():
            k = N_ABLK_FULL - 1
            pltpu.sync_copy(tt_hbm.at[:, pl.ds(k * ABLK, ABLK)],
                            in0.at[:, pl.ds(0, ABLK)])
            transpose_block(in0, tr0, 128)
            pltpu.sync_copy(tr0, out_hbm.at[pl.ds(k * 128, 128), :])

        @pl.when(wid == 1)
        def _():
            nrows = A_TAIL // 4
            pltpu.sync_copy(tail_hbm, tr1.at[pl.ds(0, nrows), :])
            pltpu.sync_copy(
                tr1.at[pl.ds(0, nrows), :],
                out_hbm.at[pl.ds((N_ABLK_FULL * ABLK) // 4, nrows), :])

    return relayout_kernel


def _make_gather_kernel():
    mesh = plsc.VectorSubcoreMesh(core_axis_name="c", subcore_axis_name="s")

    @functools.partial(
        pl.kernel,
        mesh=mesh,
        out_type=jax.ShapeDtypeStruct((OUT_ROWS, 128), jnp.float32),
        scratch_types=[
            pltpu.VMEM((CHUNK,), jnp.int32),
            pltpu.VMEM((CHUNK,), jnp.int32),
            pltpu.VMEM((CHUNK, EMBED_DIM), jnp.float32),
            pltpu.VMEM((CHUNK, EMBED_DIM), jnp.float32),
            pltpu.VMEM((G * 64, 128), jnp.float32),
            pltpu.SemaphoreType.DMA,
            pltpu.SemaphoreType.DMA,
            pltpu.SemaphoreType.DMA,
        ],
        compiler_params=pltpu.CompilerParams(
            use_tc_tiling_on_sc=False, needs_layout_passes=False),
    )
    def gather_kernel(table_hbm, idxt_hbm, out_hbm, idx0, idx1,
                      rows0, rows1, slab, sg0, sg1, sw):
        wid = lax.axis_index("s") * NUM_CORES + lax.axis_index("c")
        idxs = (idx0, idx1)
        rows = (rows0, rows1)
        sgs = (sg0, sg1)
        lanes = lax.iota(jnp.int32, 16)

        def chunk_hcg(k):
            kk = wid * CPW + k
            return kk // CGRP, kk % CGRP

        def idx_copy(k, p):
            h, cg = chunk_hcg(k)
            pltpu.sync_copy(
                idxt_hbm.at[pl.ds(h * BATCH + cg * CHUNK, CHUNK)], idxs[p])

        def gather_start(p):
            pltpu.async_copy(table_hbm.at[idxs[p]], rows[p], sgs[p])

        def gather_wait(p):
            pltpu.make_async_copy(
                table_hbm.at[pl.ds(0, CHUNK)], rows[p], sgs[p]).wait()

        def write_start(k):
            h, cg = chunk_hcg(k)
            for g in range(G):
                r0 = ((h * G + g) * (BATCH // 128) + cg * 8) * 8
                pltpu.async_copy(slab.at[pl.ds(g * 64, 64), :],
                                 out_hbm.at[pl.ds(r0, 64), :], sw)

        def write_wait():
            for g in range(G):
                pltpu.make_async_copy(slab.at[pl.ds(g * 64, 64), :],
                                      out_hbm.at[pl.ds(0, 64), :], sw).wait()

        def transpose_chunk(p):
            src = rows[p]

            @plsc.parallel_loop(0, 256, unroll=16)
            def q_body(q):
                g = q // 64
                rm = q % 64
                cc = rm // 8
                e = rm % 8
                col = jnp.full((16,), g * 8 + e, jnp.int32)
                base = cc * 128
                for t in range(8):
                    v = plsc.load_gather(src, [(base + 16 * t) + lanes, col])
                    slab[q, pl.ds(16 * t, 16)] = v

        def step(k, p, first):
            with jax.named_scope("b_gwait"):
                gather_wait(p)
            gather_start(1 - p)
            with jax.named_scope("b_idxcopy"):
                idx_copy(jnp.minimum(k + 2, CPW - 1), p)
            if not first:
                with jax.named_scope("b_wwait"):
                    write_wait()
            with jax.named_scope("b_tr"):
                transpose_chunk(p)
            write_start(k)

        idx_copy(0, 0)
        gather_start(0)
        idx_copy(1, 1)
        step(0, 0, True)

        def pair_body(k2, carry):
            k = 1 + 2 * k2
            step(k, 1, False)
            step(k + 1, 0, False)
            return carry
        lax.fori_loop(0, (CPW - 1) // 2, pair_body, 0)

        write_wait()
        gather_wait(1)

    return gather_kernel


_relayout = _make_relayout_kernel()
_gather = _make_gather_kernel()


@jax.jit
def kernel(word_indices, table):
    tt = table.T
    tail16 = lax.slice(
        table, (N_ABLK_FULL * ABLK, 0), (VOCAB, EMBED_DIM)
    ).reshape(A_TAIL // 4, 128)
    tabr = _relayout(tt, tail16)
    tab_lin = tabr.reshape(VOCAB, EMBED_DIM)
    idxt_flat = word_indices.T.reshape(TOTAL).astype(jnp.int32)
    out = _gather(tab_lin, idxt_flat)
    out = out.reshape(HIST, G, BATCH // 128, 8, 128)
    out = out.transpose(2, 4, 0, 1, 3)
    return out.reshape(BATCH, HIST, EMBED_DIM)


# speedup vs baseline: 1.1193x; 1.0189x over previous
import functools

import jax
import jax.numpy as jnp
from jax import lax
from jax.experimental import pallas as pl
from jax.experimental.pallas import tpu as pltpu
from jax.experimental.pallas import tpu_sc as plsc

VOCAB = 1000000
EMBED_DIM = 32
BATCH = 4096
HIST = 200
TOTAL = BATCH * HIST

NUM_CORES = 2
NUM_SUBCORES = 16
NW = NUM_CORES * NUM_SUBCORES

ABLK = 512
N_ABLK_FULL = VOCAB // ABLK
A_LOOP = 61
A_TAIL = VOCAB - N_ABLK_FULL * ABLK

CHUNK = 1024
CPW = TOTAL // (CHUNK * NW)
CGRP = BATCH // CHUNK
G = EMBED_DIM // 8
OUT_ROWS = TOTAL * EMBED_DIM // 128


def _make_relayout_kernel():
    mesh = plsc.VectorSubcoreMesh(core_axis_name="c", subcore_axis_name="s")

    @functools.partial(
        pl.kernel,
        mesh=mesh,
        out_type=jax.ShapeDtypeStruct((VOCAB // 4, 128), jnp.float32),
        scratch_types=[
            pltpu.VMEM((EMBED_DIM, ABLK + 8), jnp.float32),
            pltpu.VMEM((EMBED_DIM, ABLK + 8), jnp.float32),
            pltpu.VMEM((128, 128), jnp.float32),
            pltpu.VMEM((128, 128), jnp.float32),
            pltpu.SemaphoreType.DMA,
            pltpu.SemaphoreType.DMA,
            pltpu.SemaphoreType.DMA,
            pltpu.SemaphoreType.DMA,
        ],
        compiler_params=pltpu.CompilerParams(
            use_tc_tiling_on_sc=True, needs_layout_passes=False),
    )
    def relayout_kernel(tt_hbm, tail_hbm, out_hbm, in0, in1, tr0, tr1,
                        si0, si1, sw0, sw1):
        wid = lax.axis_index("s") * NUM_CORES + lax.axis_index("c")
        ins = (in0, in1)
        trs = (tr0, tr1)
        sis = (si0, si1)
        sws = (sw0, sw1)
        lanes = lax.iota(jnp.int32, 16)
        evec = [lanes + 16 * half for half in range(2)]

        def blk(j):
            return wid + NW * j

        def in_start(j, p):
            pltpu.async_copy(
                tt_hbm.at[:, pl.ds(blk(j) * ABLK, ABLK)],
                ins[p].at[:, pl.ds(0, ABLK)], sis[p])

        def in_wait(p):
            pltpu.make_async_copy(
                tt_hbm.at[:, pl.ds(0, ABLK)],
                ins[p].at[:, pl.ds(0, ABLK)], sis[p]).wait()

        def out_start(j, p):
            pltpu.async_copy(
                trs[p], out_hbm.at[pl.ds(blk(j) * 128, 128), :], sws[p])

        def out_wait(p):
            pltpu.make_async_copy(
                trs[p], out_hbm.at[pl.ds(0, 128), :], sws[p]).wait()

        def transpose_block(src, dst, nrows):
            @plsc.parallel_loop(0, nrows, unroll=8)
            def row_body(r):
                for t in range(8):
                    col = jnp.full((16,), r * 4 + t // 2, jnp.int32)
                    v = plsc.load_gather(src, [evec[t % 2], col])
                    dst[r, pl.ds(16 * t, 16)] = v

        def step(j, p, first):
            in_wait(p)
            in_start(j + 1, 1 - p)
            if not first:
                out_wait(p)
            transpose_block(ins[p], trs[p], 128)
            out_start(j, p)

        in_start(0, 0)
        step(0, 0, True)
        step(1, 1, True)

        def pair_body(j2, carry):
            j = 2 + 2 * j2
            step(j, 0, False)
            step(j + 1, 1, False)
            return carry
        lax.fori_loop(0, (A_LOOP - 3) // 2, pair_body, 0)

        in_wait(0)
        out_wait(0)
        transpose_block(in0, tr0, 128)
        out_start(A_LOOP - 1, 0)
        out_wait(1)
        out_wait(0)

        @pl.when(wid == 0)
        def _():
            k = N_ABLK_FULL - 1
            pltpu.sync_copy(tt_hbm.at[:, pl.ds(k * ABLK, ABLK)],
                            in0.at[:, pl.ds(0, ABLK)])
            transpose_block(in0, tr0, 128)
            pltpu.sync_copy(tr0, out_hbm.at[pl.ds(k * 128, 128), :])

        @pl.when(wid == 1)
        def _():
            nrows = A_TAIL // 4
            pltpu.sync_copy(tail_hbm, tr1.at[pl.ds(0, nrows), :])
            pltpu.sync_copy(
                tr1.at[pl.ds(0, nrows), :],
                out_hbm.at[pl.ds((N_ABLK_FULL * ABLK) // 4, nrows), :])

    return relayout_kernel


def _make_gather_kernel():
    mesh = plsc.VectorSubcoreMesh(core_axis_name="c", subcore_axis_name="s")

    @functools.partial(
        pl.kernel,
        mesh=mesh,
        out_type=jax.ShapeDtypeStruct((OUT_ROWS, 128), jnp.float32),
        scratch_types=[
            pltpu.VMEM((CHUNK,), jnp.int32),
            pltpu.VMEM((CHUNK,), jnp.int32),
            pltpu.VMEM((CHUNK, EMBED_DIM), jnp.float32),
            pltpu.VMEM((CHUNK, EMBED_DIM), jnp.float32),
            pltpu.VMEM((G * 64, 128), jnp.float32),
            pltpu.SemaphoreType.DMA,
            pltpu.SemaphoreType.DMA,
            pltpu.SemaphoreType.DMA,
        ],
        compiler_params=pltpu.CompilerParams(
            use_tc_tiling_on_sc=False, needs_layout_passes=False),
    )
    def gather_kernel(table_hbm, idxt_hbm, out_hbm, idx0, idx1,
                      rows0, rows1, slab, sg0, sg1, sw):
        wid = lax.axis_index("s") * NUM_CORES + lax.axis_index("c")
        idxs = (idx0, idx1)
        rows = (rows0, rows1)
        sgs = (sg0, sg1)
        lanes = lax.iota(jnp.int32, 16)

        def chunk_hcg(k):
            kk = wid * CPW + k
            return kk // CGRP, kk % CGRP

        def idx_copy(k, p):
            h, cg = chunk_hcg(k)
            pltpu.sync_copy(
                idxt_hbm.at[pl.ds(h * BATCH + cg * CHUNK, CHUNK)], idxs[p])

        def gather_start(p):
            pltpu.async_copy(table_hbm.at[idxs[p]], rows[p], sgs[p])

        def gather_wait(p):
            pltpu.make_async_copy(
                table_hbm.at[pl.ds(0, CHUNK)], rows[p], sgs[p]).wait()

        def write_start(k):
            h, cg = chunk_hcg(k)
            for g in range(G):
                r0 = ((h * G + g) * (BATCH // 128) + cg * 8) * 8
                pltpu.async_copy(slab.at[pl.ds(g * 64, 64), :],
                                 out_hbm.at[pl.ds(r0, 64), :], sw)

        def write_wait():
            for g in range(G):
                pltpu.make_async_copy(slab.at[pl.ds(g * 64, 64), :],
                                      out_hbm.at[pl.ds(0, 64), :], sw).wait()

        def transpose_chunk(p):
            src = rows[p]

            @plsc.parallel_loop(0, 256, unroll=8)
            def q_body(q):
                g = q // 64
                rm = q % 64
                cc = rm // 8
                e = rm % 8
                col = jnp.full((16,), g * 8 + e, jnp.int32)
                base = cc * 128
                for t in range(8):
                    v = plsc.load_gather(src, [(base + 16 * t) + lanes, col])
                    slab[q, pl.ds(16 * t, 16)] = v

        def step(k, p, first):
            gather_wait(p)
            gather_start(1 - p)
            idx_copy(jnp.minimum(k + 2, CPW - 1), p)
            if not first:
                write_wait()
            transpose_chunk(p)
            write_start(k)

        idx_copy(0, 0)
        gather_start(0)
        idx_copy(1, 1)
        step(0, 0, True)

        def pair_body(k2, carry):
            k = 1 + 2 * k2
            step(k, 1, False)
            step(k + 1, 0, False)
            return carry
        lax.fori_loop(0, (CPW - 1) // 2, pair_body, 0)

        write_wait()
        gather_wait(1)

    return gather_kernel


_relayout = _make_relayout_kernel()
_gather = _make_gather_kernel()


@jax.jit
def kernel(word_indices, table):
    tt = table.T
    tail16 = lax.slice(
        table, (N_ABLK_FULL * ABLK, 0), (VOCAB, EMBED_DIM)
    ).reshape(A_TAIL // 4, 128)
    tabr = _relayout(tt, tail16)
    tab_lin = tabr.reshape(VOCAB, EMBED_DIM)
    idxt_flat = word_indices.T.reshape(TOTAL).astype(jnp.int32)
    out = _gather(tab_lin, idxt_flat)
    out = out.reshape(HIST, G, BATCH // 128, 8, 128)
    out = out.transpose(2, 4, 0, 1, 3)
    return out.reshape(BATCH, HIST, EMBED_DIM)
